# Initial kernel scaffold; baseline (speedup 1.0000x reference)
import jax
import jax.numpy as jnp
from jax.experimental import pallas as pl


def kernel(x_s, x_d, edge_index, edge_attr, F0, F1, F2, W1, b1, W2, b2):
    raise NotImplementedError("write your pallas kernel here")


# speedup vs baseline: 1.1001x; 1.1001x over previous
import functools

import jax
import jax.numpy as jnp
from jax.experimental import pallas as pl
from jax.experimental.pallas import tpu as pltpu

N = 10000
E = 320000
DD = 64
H = 128


def _mlp_body(g_ref, ea_ref, w1e_ref, b1_ref, w2t_ref, b2_ref, out_ref):
    pre = g_ref[...] + jnp.dot(ea_ref[...], w1e_ref[...],
                               preferred_element_type=jnp.float32) + b1_ref[...]
    h = jnp.maximum(pre, 0.0)
    s = jnp.dot(h, w2t_ref[...], preferred_element_type=jnp.float32) + b2_ref[...]
    nrm = jnp.sqrt(jnp.sum(s * s, axis=1, keepdims=True))
    s = s / nrm
    s = jnp.where(jnp.isnan(s), 0.0, s)
    out_ref[...] = s


def _edge_mlp(G, edge_attr, W1e_t, b1, W2_t, b2, block=2560):
    grid = E // block
    return pl.pallas_call(
        _mlp_body,
        grid=(grid,),
        in_specs=[
            pl.BlockSpec((block, H), lambda i: (i, 0)),
            pl.BlockSpec((block, 16), lambda i: (i, 0)),
            pl.BlockSpec((16, H), lambda i: (0, 0)),
            pl.BlockSpec((1, H), lambda i: (0, 0)),
            pl.BlockSpec((H, DD), lambda i: (0, 0)),
            pl.BlockSpec((1, DD), lambda i: (0, 0)),
        ],
        out_specs=pl.BlockSpec((block, DD), lambda i: (i, 0)),
        out_shape=jax.ShapeDtypeStruct((E, DD), jnp.float32),
    )(G, edge_attr, W1e_t, b1, W2_t, b2)


def kernel(x_s, x_d, edge_index, edge_attr, F0, F1, F2, W1, b1, W2, b2):
    row = edge_index[0]
    col = edge_index[1]
    W_sr = W1[:, 0:64]
    W_sc = W1[:, 64:128]
    W_dr = W1[:, 128:192]
    W_dc = W1[:, 192:256]
    W_e = W1[:, 256:272]
    P_r = x_s @ W_sr.T + x_d @ W_dr.T
    P_c = x_s @ W_sc.T + x_d @ W_dc.T

    G = P_r[row] + P_c[col]
    s_full = _edge_mlp(G, edge_attr, W_e.T, b1[None, :], W2.T, b2[None, :])

    out = x_d @ F0.T
    filters = (F1, F2)
    for k in range(2):
        mask = jnp.sum(out, axis=1) != 0
        em = (mask[row] | mask[col]).astype(out.dtype)
        grad = out[col] - out[row]
        shift = grad * s_full * em[:, None]
        scattered = jax.ops.segment_sum(shift, col, num_segments=N)
        out = out + scattered @ filters[k].T
    return out


# speedup vs baseline: 1.4782x; 1.3437x over previous
import functools

import jax
import jax.numpy as jnp
from jax import lax
from jax.experimental import pallas as pl
from jax.experimental.pallas import tpu as pltpu
from jax.experimental.pallas import tpu_sc as plsc

N = 10000
E = 320000
DD = 64
H = 128

_NC = 2
_NS = 16
_NW = _NC * _NS
_EPW = E // _NW
_GW = 80
_GWIN = _EPW // _GW


def _gather_pre_body(pr_hbm, pc_hbm, row_hbm, col_hbm, g_hbm,
                     idxr_v, idxc_v, buf_v, sem):
    wid = lax.axis_index("s") * _NC + lax.axis_index("c")
    base = wid * _EPW

    def step(w, _):
        off = base + w * _GW
        pltpu.sync_copy(row_hbm.at[pl.ds(off, _GW)], idxr_v)
        pltpu.sync_copy(col_hbm.at[pl.ds(off, _GW)], idxc_v)
        pltpu.async_copy(pr_hbm.at[idxr_v], buf_v, sem).wait()
        pltpu.async_copy(pc_hbm.at[idxc_v], buf_v, sem, add=True).wait()
        pltpu.sync_copy(buf_v, g_hbm.at[pl.ds(off, _GW)])
        return _

    lax.fori_loop(0, _GWIN, step, 0)


def _gather_pre(P_r, P_c, row, col):
    mesh = plsc.VectorSubcoreMesh(core_axis_name="c", subcore_axis_name="s")
    return pl.kernel(
        _gather_pre_body,
        out_type=jax.ShapeDtypeStruct((E, H), jnp.float32),
        mesh=mesh,
        scratch_types=[
            pltpu.VMEM((_GW,), jnp.int32),
            pltpu.VMEM((_GW,), jnp.int32),
            pltpu.VMEM((_GW, H), jnp.float32),
            pltpu.SemaphoreType.DMA,
        ],
    )(P_r, P_c, row, col)


def _mlp_body(g_ref, ea_ref, w1e_ref, b1_ref, w2t_ref, b2_ref, out_ref):
    pre = g_ref[...] + jnp.dot(ea_ref[...], w1e_ref[...],
                               preferred_element_type=jnp.float32) + b1_ref[...]
    h = jnp.maximum(pre, 0.0)
    s = jnp.dot(h, w2t_ref[...], preferred_element_type=jnp.float32) + b2_ref[...]
    nrm = jnp.sqrt(jnp.sum(s * s, axis=1, keepdims=True))
    s = s / nrm
    s = jnp.where(jnp.isnan(s), 0.0, s)
    out_ref[...] = s


def _edge_mlp(G, edge_attr, W1e_t, b1, W2_t, b2, block=2560):
    grid = E // block
    return pl.pallas_call(
        _mlp_body,
        grid=(grid,),
        in_specs=[
            pl.BlockSpec((block, H), lambda i: (i, 0)),
            pl.BlockSpec((block, 16), lambda i: (i, 0)),
            pl.BlockSpec((16, H), lambda i: (0, 0)),
            pl.BlockSpec((1, H), lambda i: (0, 0)),
            pl.BlockSpec((H, DD), lambda i: (0, 0)),
            pl.BlockSpec((1, DD), lambda i: (0, 0)),
        ],
        out_specs=pl.BlockSpec((block, DD), lambda i: (i, 0)),
        out_shape=jax.ShapeDtypeStruct((E, DD), jnp.float32),
    )(G, edge_attr, W1e_t, b1, W2_t, b2)


def kernel(x_s, x_d, edge_index, edge_attr, F0, F1, F2, W1, b1, W2, b2):
    row = edge_index[0]
    col = edge_index[1]
    W_sr = W1[:, 0:64]
    W_sc = W1[:, 64:128]
    W_dr = W1[:, 128:192]
    W_dc = W1[:, 192:256]
    W_e = W1[:, 256:272]
    P_r = x_s @ W_sr.T + x_d @ W_dr.T
    P_c = x_s @ W_sc.T + x_d @ W_dc.T

    G = _gather_pre(P_r, P_c, row, col)
    s_full = _edge_mlp(G, edge_attr, W_e.T, b1[None, :], W2.T, b2[None, :])

    out = x_d @ F0.T
    filters = (F1, F2)
    for k in range(2):
        mask = jnp.sum(out, axis=1) != 0
        em = (mask[row] | mask[col]).astype(out.dtype)
        grad = out[col] - out[row]
        shift = grad * s_full * em[:, None]
        scattered = jax.ops.segment_sum(shift, col, num_segments=N)
        out = out + scattered @ filters[k].T
    return out
